# Initial kernel scaffold; baseline (speedup 1.0000x reference)
import functools

import jax
import jax.numpy as jnp
from jax.experimental import pallas as pl
from jax.experimental.pallas import tpu as pltpu

_N = 16384
_D = 1024
_DQ = 128
_BLK = 2048


def _scores_body(key_feat_ref, wq_ref, bq_ref, feats_ref, s_ref):
    q_key = jnp.dot(key_feat_ref[...], wq_ref[...]) + bq_ref[...]
    q = jnp.dot(feats_ref[...], wq_ref[...]) + bq_ref[...]
    s_ref[...] = jnp.dot(q, q_key.T)


def _scores(feats, key_feat, Wq, bq):
    bq2 = bq.reshape(1, _DQ)
    return pl.pallas_call(
        _scores_body,
        grid=(_N // _BLK,),
        in_specs=[
            pl.BlockSpec((1, _D), lambda i: (0, 0)),
            pl.BlockSpec((_D, _DQ), lambda i: (0, 0)),
            pl.BlockSpec((1, _DQ), lambda i: (0, 0)),
            pl.BlockSpec((_BLK, _D), lambda i: (i, 0)),
        ],
        out_specs=pl.BlockSpec((_BLK, 1), lambda i: (i, 0)),
        out_shape=jax.ShapeDtypeStruct((_N, 1), jnp.float32),
    )(key_feat, Wq, bq2, feats)


def kernel(feats, key_feat, Wq, bq, Wv, bv, top_k):
    s = _scores(feats, key_feat, Wq, bq)
    A = jax.nn.softmax(s / jnp.sqrt(jnp.asarray(_DQ, dtype=jnp.float32)), axis=0)
    _, idx = jax.lax.top_k(A.T, _D)
    idx_flat = idx.reshape(-1)
    idx_flat = idx_flat + (jnp.asarray(top_k) - _D).astype(idx_flat.dtype)
    top_k_features = jnp.take(feats, idx_flat, axis=0)
    fusion_feature = (A.T @ feats) @ Wv + bv
    return (top_k_features, fusion_feature)


# speedup vs baseline: 1.8283x; 1.8283x over previous
import functools

import jax
import jax.numpy as jnp
import numpy as np
from jax import lax
from jax.experimental import pallas as pl
from jax.experimental.pallas import tpu as pltpu
from jax.experimental.pallas import tpu_sc as plsc

_N = 16384
_D = 1024
_DQ = 128
_KC = 256
_RSCALE = np.float32(1.0) / np.sqrt(np.float32(128.0))

_bf16 = jnp.bfloat16
_f32 = jnp.float32


def _qk_body(kf_ref, wq_ref, bq_ref, qkb_ref):
    qk = jnp.dot(kf_ref[...], wq_ref[...]) + bq_ref[...]
    qkb_ref[...] = qk.astype(_bf16)


def _qk(key_feat, Wq, bq2):
    return pl.pallas_call(
        _qk_body,
        in_specs=[
            pl.BlockSpec((1, _D), lambda: (0, 0)),
            pl.BlockSpec((_D, _DQ), lambda: (0, 0)),
            pl.BlockSpec((1, _DQ), lambda: (0, 0)),
        ],
        out_specs=pl.BlockSpec((1, _DQ), lambda: (0, 0)),
        out_shape=jax.ShapeDtypeStruct((1, _DQ), _bf16),
    )(key_feat, Wq, bq2)


_MBLK = 2048


def _score_body(qkb_ref, wq_ref, bq_ref, f_ref, s_ref):
    acc = jnp.zeros((_MBLK, _DQ), _f32)
    for j in range(_D // _KC):
        acc = acc + jnp.dot(f_ref[:, j * _KC:(j + 1) * _KC],
                            wq_ref[j * _KC:(j + 1) * _KC, :])
    qb = (acc + bq_ref[...]).astype(_bf16)
    s_ref[...] = jnp.dot(qb, qkb_ref[...].T, preferred_element_type=_f32)


def _scores(qkb, Wq, bq2, feats):
    return pl.pallas_call(
        _score_body,
        grid=(_N // _MBLK,),
        in_specs=[
            pl.BlockSpec((1, _DQ), lambda i: (0, 0)),
            pl.BlockSpec((_D, _DQ), lambda i: (0, 0)),
            pl.BlockSpec((1, _DQ), lambda i: (0, 0)),
            pl.BlockSpec((_MBLK, _D), lambda i: (i, 0)),
        ],
        out_specs=pl.BlockSpec((_MBLK, 1), lambda i: (i, 0)),
        out_shape=jax.ShapeDtypeStruct((_N, 1), _f32),
    )(qkb, Wq, bq2, feats)


def _t_i32(x):
    return lax.bitcast_convert_type(
        lax.bitcast_convert_type(x, _f32).T, jnp.int32)


def _bitonic_topk(key, idx, trans):
    for p in range(1, 15):
        for j in reversed(range(p)):
            need_trans = j >= 7
            if need_trans != trans:
                key = key.T
                idx = _t_i32(idx)
                trans = need_trans
            sh = (1 << j) if j < 7 else (1 << (j - 7))
            io0 = lax.broadcasted_iota(jnp.int32, (128, 128), 0)
            up = (io0 & sh) == 0
            if p < 7:
                ax = 1 if trans else 0
                bit = 1 << p
            else:
                ax = 0 if trans else 1
                bit = 1 << (p - 7)
            iod = lax.broadcasted_iota(jnp.int32, (128, 128), ax)
            dirm = (iod & bit) == 0
            pk = jnp.where(up, jnp.roll(key, -sh, axis=0), jnp.roll(key, sh, axis=0))
            pi = jnp.where(up, jnp.roll(idx, -sh, axis=0), jnp.roll(idx, sh, axis=0))
            better = (key > pk) | ((key == pk) & (idx < pi))
            keep = (up == better) == dirm
            key = jnp.where(keep, key, pk)
            idx = jnp.where(keep, idx, pi)
    if trans:
        key = key.T
        idx = _t_i32(idx)
    return key, idx


def _sort_body(s_ref, tko_ref, at_ref, idx_ref):
    v = s_ref[...]
    x = v * _RSCALE
    m = jnp.max(x)
    e = jnp.exp(x - m)
    z = jnp.sum(e)
    a = e / z
    at_ref[...] = a
    hi_io = lax.broadcasted_iota(jnp.int32, (128, 128), 0)
    lo_io = lax.broadcasted_iota(jnp.int32, (128, 128), 1)
    n = 128 * hi_io + lo_io
    _, idx_sorted = _bitonic_topk(a, n, True)
    top = idx_sorted[:, 0:8]
    idx_ref[...] = _t_i32(top) + tko_ref[0]


def _sort_topk(s2d, tko):
    return pl.pallas_call(
        _sort_body,
        in_specs=[
            pl.BlockSpec((128, 128), lambda: (0, 0)),
            pl.BlockSpec(memory_space=pltpu.SMEM),
        ],
        out_specs=[
            pl.BlockSpec((128, 128), lambda: (0, 0)),
            pl.BlockSpec((8, 128), lambda: (0, 0)),
        ],
        out_shape=[
            jax.ShapeDtypeStruct((128, 128), _f32),
            jax.ShapeDtypeStruct((8, 128), jnp.int32),
        ],
    )(s2d, tko)


_NW = 32
_BPW = 1024 // _NW


@functools.lru_cache(maxsize=1)
def _make_gather_sc():
    @functools.partial(
        pl.kernel,
        out_type=jax.ShapeDtypeStruct((1024, _D), _f32),
        mesh=plsc.VectorSubcoreMesh(core_axis_name="c", subcore_axis_name="s"),
        scratch_types=[
            pltpu.VMEM((_BPW,), jnp.int32),
            pltpu.VMEM((_BPW, _D), _f32),
            pltpu.SemaphoreType.DMA,
        ],
    )
    def _gather_sc(feats_hbm, idx_hbm, out_hbm, idx_v, rows_v, sem):
        wid = lax.axis_index("s") * 2 + lax.axis_index("c")
        base = wid * _BPW
        pltpu.sync_copy(idx_hbm.at[pl.ds(base, _BPW)], idx_v)
        pltpu.async_copy(feats_hbm.at[idx_v], rows_v, sem).wait()
        pltpu.sync_copy(rows_v, out_hbm.at[pl.ds(base, _BPW)])

    return _gather_sc


def _u_body(at_ref, f_ref, u_ref):
    i = pl.program_id(0)

    @pl.when(i == 0)
    def _init():
        u_ref[...] = jnp.zeros_like(u_ref)

    bt = at_ref[...]
    acc = u_ref[...]
    for t in range(16):
        acc = acc + jnp.dot(bt[t:t + 1, :], f_ref[128 * t:128 * (t + 1), :],
                            precision="highest", preferred_element_type=_f32)
    u_ref[...] = acc


def _u(a2d, feats):
    return pl.pallas_call(
        _u_body,
        grid=(_N // 2048,),
        in_specs=[
            pl.BlockSpec((16, 128), lambda i: (i, 0)),
            pl.BlockSpec((2048, _D), lambda i: (i, 0)),
        ],
        out_specs=pl.BlockSpec((1, _D), lambda i: (0, 0)),
        out_shape=jax.ShapeDtypeStruct((1, _D), _f32),
    )(a2d, feats)


def _fusion_body(u_ref, wv_ref, bv_ref, out_ref):
    out_ref[...] = jnp.dot(u_ref[...], wv_ref[...], precision="highest",
                           preferred_element_type=_f32) + bv_ref[...]


def _fusion(u, Wv, bv2):
    return pl.pallas_call(
        _fusion_body,
        in_specs=[
            pl.BlockSpec((1, _D), lambda: (0, 0)),
            pl.BlockSpec((_D, _D), lambda: (0, 0)),
            pl.BlockSpec((1, _D), lambda: (0, 0)),
        ],
        out_specs=pl.BlockSpec((1, _D), lambda: (0, 0)),
        out_shape=jax.ShapeDtypeStruct((1, _D), _f32),
    )(u, Wv, bv2)


def kernel(feats, key_feat, Wq, bq, Wv, bv, top_k):
    bq2 = bq.reshape(1, _DQ)
    qkb = _qk(key_feat, Wq, bq2)
    s = _scores(qkb, Wq, bq2, feats)
    s2d = s.reshape(128, 128)
    tko = (jnp.asarray(top_k, jnp.int32) - jnp.int32(_D)).reshape(1)
    a2d, idx8 = _sort_topk(s2d, tko)
    idx_flat = idx8.reshape(-1)
    top_k_features = _make_gather_sc()(feats, idx_flat)
    u = _u(a2d, feats)
    fusion_feature = _fusion(u, Wv, bv.reshape(1, _D))
    return (top_k_features, fusion_feature)


# speedup vs baseline: 2.0580x; 1.1256x over previous
import functools

import jax
import jax.numpy as jnp
import numpy as np
from jax import lax
from jax.experimental import pallas as pl
from jax.experimental.pallas import tpu as pltpu
from jax.experimental.pallas import tpu_sc as plsc

_N = 16384
_D = 1024
_DQ = 128
_KC = 256
_RSCALE = np.float32(1.0) / np.sqrt(np.float32(128.0))

_bf16 = jnp.bfloat16
_f32 = jnp.float32


def _qk_body(kf_ref, wq_ref, bq_ref, qkb_ref):
    qk = jnp.dot(kf_ref[...], wq_ref[...]) + bq_ref[...]
    qkb_ref[...] = qk.astype(_bf16)


def _qk(key_feat, Wq, bq2):
    return pl.pallas_call(
        _qk_body,
        in_specs=[
            pl.BlockSpec((1, _D), lambda: (0, 0)),
            pl.BlockSpec((_D, _DQ), lambda: (0, 0)),
            pl.BlockSpec((1, _DQ), lambda: (0, 0)),
        ],
        out_specs=pl.BlockSpec((1, _DQ), lambda: (0, 0)),
        out_shape=jax.ShapeDtypeStruct((1, _DQ), _bf16),
    )(key_feat, Wq, bq2)


_MBLK = 2048


def _score_body(qkb_ref, wq_ref, bq_ref, f_ref, s_ref, u_ref, m_sc):
    i = pl.program_id(0)
    acc = jnp.zeros((_MBLK, _DQ), _f32)
    for j in range(_D // _KC):
        acc = acc + jnp.dot(f_ref[:, j * _KC:(j + 1) * _KC],
                            wq_ref[j * _KC:(j + 1) * _KC, :])
    qb = (acc + bq_ref[...]).astype(_bf16)
    s_blk = jnp.dot(qb, qkb_ref[...].T, preferred_element_type=_f32)
    s_ref[...] = s_blk

    x = s_blk * _RSCALE
    bmax = jnp.max(x)

    @pl.when(i == 0)
    def _init():
        m_sc[0] = -jnp.inf
        u_ref[...] = jnp.zeros_like(u_ref)

    m_old = m_sc[0]
    m_new = jnp.maximum(m_old, bmax)
    e = jnp.exp(x - m_new)
    part = lax.dot_general(e, f_ref[...],
                           dimension_numbers=(((0,), (0,)), ((), ())),
                           precision="highest",
                           preferred_element_type=_f32)
    u_ref[...] = u_ref[...] * jnp.exp(m_old - m_new) + part
    m_sc[0] = m_new


def _scores(qkb, Wq, bq2, feats):
    return pl.pallas_call(
        _score_body,
        grid=(_N // _MBLK,),
        in_specs=[
            pl.BlockSpec((1, _DQ), lambda i: (0, 0)),
            pl.BlockSpec((_D, _DQ), lambda i: (0, 0)),
            pl.BlockSpec((1, _DQ), lambda i: (0, 0)),
            pl.BlockSpec((_MBLK, _D), lambda i: (i, 0)),
        ],
        out_specs=[
            pl.BlockSpec((_MBLK, 1), lambda i: (i, 0)),
            pl.BlockSpec((1, _D), lambda i: (0, 0)),
        ],
        out_shape=[
            jax.ShapeDtypeStruct((_N, 1), _f32),
            jax.ShapeDtypeStruct((1, _D), _f32),
        ],
        scratch_shapes=[pltpu.SMEM((1,), _f32)],
    )(qkb, Wq, bq2, feats)


def _t_i32(x):
    return lax.bitcast_convert_type(
        lax.bitcast_convert_type(x, _f32).T, jnp.int32)


def _bitonic_topk(key, idx, trans):
    for p in range(1, 15):
        for j in reversed(range(p)):
            need_trans = j >= 7
            if need_trans != trans:
                key = key.T
                idx = _t_i32(idx)
                trans = need_trans
            sh = (1 << j) if j < 7 else (1 << (j - 7))
            io0 = lax.broadcasted_iota(jnp.int32, (128, 128), 0)
            up = (io0 & sh) == 0
            if p < 7:
                ax = 1 if trans else 0
                bit = 1 << p
            else:
                ax = 0 if trans else 1
                bit = 1 << (p - 7)
            iod = lax.broadcasted_iota(jnp.int32, (128, 128), ax)
            dirm = (iod & bit) == 0
            pk = jnp.where(up, jnp.roll(key, -sh, axis=0), jnp.roll(key, sh, axis=0))
            pi = jnp.where(up, jnp.roll(idx, -sh, axis=0), jnp.roll(idx, sh, axis=0))
            better = (key > pk) | ((key == pk) & (idx < pi))
            keep = (up == better) == dirm
            key = jnp.where(keep, key, pk)
            idx = jnp.where(keep, idx, pi)
    if trans:
        key = key.T
        idx = _t_i32(idx)
    return key, idx


def _sort_body(s_ref, tko_ref, z_ref, idx_ref):
    v = s_ref[...]
    x = v * _RSCALE
    m = jnp.max(x)
    e = jnp.exp(x - m)
    z = jnp.sum(e)
    a = e / z
    z_ref[...] = z * jnp.ones((1, 1), _f32)
    hi_io = lax.broadcasted_iota(jnp.int32, (128, 128), 0)
    lo_io = lax.broadcasted_iota(jnp.int32, (128, 128), 1)
    n = 128 * hi_io + lo_io
    _, idx_sorted = _bitonic_topk(a, n, True)
    top = idx_sorted[:, 0:8]
    idx_ref[...] = _t_i32(top) + tko_ref[0]


def _sort_topk(s2d, tko):
    return pl.pallas_call(
        _sort_body,
        in_specs=[
            pl.BlockSpec((128, 128), lambda: (0, 0)),
            pl.BlockSpec(memory_space=pltpu.SMEM),
        ],
        out_specs=[
            pl.BlockSpec((1, 1), lambda: (0, 0)),
            pl.BlockSpec((8, 128), lambda: (0, 0)),
        ],
        out_shape=[
            jax.ShapeDtypeStruct((1, 1), _f32),
            jax.ShapeDtypeStruct((8, 128), jnp.int32),
        ],
    )(s2d, tko)


_NW = 32
_BPW = 1024 // _NW


@functools.lru_cache(maxsize=1)
def _make_gather_sc():
    @functools.partial(
        pl.kernel,
        out_type=jax.ShapeDtypeStruct((1024, _D), _f32),
        mesh=plsc.VectorSubcoreMesh(core_axis_name="c", subcore_axis_name="s"),
        scratch_types=[
            pltpu.VMEM((_BPW,), jnp.int32),
            pltpu.VMEM((_BPW, _D), _f32),
            pltpu.SemaphoreType.DMA,
        ],
    )
    def _gather_sc(feats_hbm, idx_hbm, out_hbm, idx_v, rows_v, sem):
        wid = lax.axis_index("s") * 2 + lax.axis_index("c")
        base = wid * _BPW
        pltpu.sync_copy(idx_hbm.at[pl.ds(base, _BPW)], idx_v)
        pltpu.async_copy(feats_hbm.at[idx_v], rows_v, sem).wait()
        pltpu.sync_copy(rows_v, out_hbm.at[pl.ds(base, _BPW)])

    return _gather_sc


def _fusion_body(u_ref, z_ref, wv_ref, bv_ref, out_ref):
    a_row = u_ref[...] / z_ref[...]
    out_ref[...] = jnp.dot(a_row, wv_ref[...], precision="highest",
                           preferred_element_type=_f32) + bv_ref[...]


def _fusion(u, z, Wv, bv2):
    return pl.pallas_call(
        _fusion_body,
        in_specs=[
            pl.BlockSpec((1, _D), lambda: (0, 0)),
            pl.BlockSpec((1, 1), lambda: (0, 0)),
            pl.BlockSpec((_D, _D), lambda: (0, 0)),
            pl.BlockSpec((1, _D), lambda: (0, 0)),
        ],
        out_specs=pl.BlockSpec((1, _D), lambda: (0, 0)),
        out_shape=jax.ShapeDtypeStruct((1, _D), _f32),
    )(u, z, Wv, bv2)


def kernel(feats, key_feat, Wq, bq, Wv, bv, top_k):
    bq2 = bq.reshape(1, _DQ)
    qkb = _qk(key_feat, Wq, bq2)
    s, u = _scores(qkb, Wq, bq2, feats)
    s2d = s.reshape(128, 128)
    tko = (jnp.asarray(top_k, jnp.int32) - jnp.int32(_D)).reshape(1)
    z, idx8 = _sort_topk(s2d, tko)
    idx_flat = idx8.reshape(-1)
    top_k_features = _make_gather_sc()(feats, idx_flat)
    fusion_feature = _fusion(u, z, Wv, bv.reshape(1, _D))
    return (top_k_features, fusion_feature)


# speedup vs baseline: 2.0738x; 1.0077x over previous
import functools

import jax
import jax.numpy as jnp
import numpy as np
from jax import lax
from jax.experimental import pallas as pl
from jax.experimental.pallas import tpu as pltpu
from jax.experimental.pallas import tpu_sc as plsc

_N = 16384
_D = 1024
_DQ = 128
_KC = 256
_RSCALE = np.float32(1.0) / np.sqrt(np.float32(128.0))

_bf16 = jnp.bfloat16
_f32 = jnp.float32


_MBLK = 2048


def _score_body(kf_ref, wq_ref, bq_ref, f_ref, s_ref, u_ref, qkb_sc, m_sc):
    i = pl.program_id(0)

    @pl.when(i == 0)
    def _qk():
        qk = jnp.dot(kf_ref[...], wq_ref[...]) + bq_ref[...]
        qkb_sc[...] = qk.astype(_bf16)

    acc = jnp.zeros((_MBLK, _DQ), _f32)
    for j in range(_D // _KC):
        acc = acc + jnp.dot(f_ref[:, j * _KC:(j + 1) * _KC],
                            wq_ref[j * _KC:(j + 1) * _KC, :])
    qb = (acc + bq_ref[...]).astype(_bf16)
    s_blk = jnp.dot(qb, qkb_sc[...].T, preferred_element_type=_f32)
    s_ref[...] = s_blk

    x = s_blk * _RSCALE
    bmax = jnp.max(x)

    @pl.when(i == 0)
    def _init():
        m_sc[0] = -jnp.inf
        u_ref[...] = jnp.zeros_like(u_ref)

    m_old = m_sc[0]
    m_new = jnp.maximum(m_old, bmax)
    e = jnp.exp(x - m_new)
    part = lax.dot_general(e, f_ref[...],
                           dimension_numbers=(((0,), (0,)), ((), ())),
                           precision="highest",
                           preferred_element_type=_f32)
    u_ref[...] = u_ref[...] * jnp.exp(m_old - m_new) + part
    m_sc[0] = m_new


def _scores(key_feat, Wq, bq2, feats):
    return pl.pallas_call(
        _score_body,
        grid=(_N // _MBLK,),
        in_specs=[
            pl.BlockSpec((1, _D), lambda i: (0, 0)),
            pl.BlockSpec((_D, _DQ), lambda i: (0, 0)),
            pl.BlockSpec((1, _DQ), lambda i: (0, 0)),
            pl.BlockSpec((_MBLK, _D), lambda i: (i, 0)),
        ],
        out_specs=[
            pl.BlockSpec((_MBLK, 1), lambda i: (i, 0)),
            pl.BlockSpec((1, _D), lambda i: (0, 0)),
        ],
        out_shape=[
            jax.ShapeDtypeStruct((_N, 1), _f32),
            jax.ShapeDtypeStruct((1, _D), _f32),
        ],
        scratch_shapes=[pltpu.VMEM((1, _DQ), _bf16), pltpu.SMEM((1,), _f32)],
    )(key_feat, Wq, bq2, feats)


def _t_i32(x):
    return lax.bitcast_convert_type(
        lax.bitcast_convert_type(x, _f32).T, jnp.int32)


def _bitonic_topk(key, idx, trans):
    for p in range(1, 15):
        for j in reversed(range(p)):
            need_trans = j >= 7
            if need_trans != trans:
                key = key.T
                idx = _t_i32(idx)
                trans = need_trans
            sh = (1 << j) if j < 7 else (1 << (j - 7))
            io0 = lax.broadcasted_iota(jnp.int32, (128, 128), 0)
            up = (io0 & sh) == 0
            if p < 7:
                ax = 1 if trans else 0
                bit = 1 << p
            else:
                ax = 0 if trans else 1
                bit = 1 << (p - 7)
            iod = lax.broadcasted_iota(jnp.int32, (128, 128), ax)
            dirm = (iod & bit) == 0
            pk = jnp.where(up, jnp.roll(key, -sh, axis=0), jnp.roll(key, sh, axis=0))
            pi = jnp.where(up, jnp.roll(idx, -sh, axis=0), jnp.roll(idx, sh, axis=0))
            better = (key > pk) | ((key == pk) & (idx < pi))
            keep = (up == better) == dirm
            key = jnp.where(keep, key, pk)
            idx = jnp.where(keep, idx, pi)
    if trans:
        key = key.T
        idx = _t_i32(idx)
    return key, idx


def _sort_body(s_ref, tko_ref, u_ref, wv_ref, bv_ref, fus_ref, idx_ref):
    v = s_ref[...]
    x = v * _RSCALE
    m = jnp.max(x)
    e = jnp.exp(x - m)
    z = jnp.sum(e)
    a = e / z
    a_row = u_ref[...] / z
    fus_ref[...] = jnp.dot(a_row, wv_ref[...], precision="highest",
                           preferred_element_type=_f32) + bv_ref[...]
    hi_io = lax.broadcasted_iota(jnp.int32, (128, 128), 0)
    lo_io = lax.broadcasted_iota(jnp.int32, (128, 128), 1)
    n = 128 * hi_io + lo_io
    _, idx_sorted = _bitonic_topk(a, n, True)
    top = idx_sorted[:, 0:8]
    idx_ref[...] = _t_i32(top) + tko_ref[0]


def _sort_topk_fusion(s2d, tko, u, Wv, bv2):
    return pl.pallas_call(
        _sort_body,
        in_specs=[
            pl.BlockSpec((128, 128), lambda: (0, 0)),
            pl.BlockSpec(memory_space=pltpu.SMEM),
            pl.BlockSpec((1, _D), lambda: (0, 0)),
            pl.BlockSpec((_D, _D), lambda: (0, 0)),
            pl.BlockSpec((1, _D), lambda: (0, 0)),
        ],
        out_specs=[
            pl.BlockSpec((1, _D), lambda: (0, 0)),
            pl.BlockSpec((8, 128), lambda: (0, 0)),
        ],
        out_shape=[
            jax.ShapeDtypeStruct((1, _D), _f32),
            jax.ShapeDtypeStruct((8, 128), jnp.int32),
        ],
    )(s2d, tko, u, Wv, bv2)


_NW = 32
_BPW = 1024 // _NW


@functools.lru_cache(maxsize=1)
def _make_gather_sc():
    @functools.partial(
        pl.kernel,
        out_type=jax.ShapeDtypeStruct((1024, _D), _f32),
        mesh=plsc.VectorSubcoreMesh(core_axis_name="c", subcore_axis_name="s"),
        scratch_types=[
            pltpu.VMEM((_BPW,), jnp.int32),
            pltpu.VMEM((_BPW, _D), _f32),
            pltpu.SemaphoreType.DMA,
        ],
    )
    def _gather_sc(feats_hbm, idx_hbm, out_hbm, idx_v, rows_v, sem):
        wid = lax.axis_index("s") * 2 + lax.axis_index("c")
        base = wid * _BPW
        pltpu.sync_copy(idx_hbm.at[pl.ds(base, _BPW)], idx_v)
        pltpu.async_copy(feats_hbm.at[idx_v], rows_v, sem).wait()
        pltpu.sync_copy(rows_v, out_hbm.at[pl.ds(base, _BPW)])

    return _gather_sc


def kernel(feats, key_feat, Wq, bq, Wv, bv, top_k):
    bq2 = bq.reshape(1, _DQ)
    s, u = _scores(key_feat, Wq, bq2, feats)
    s2d = s.reshape(128, 128)
    tko = (jnp.asarray(top_k, jnp.int32) - jnp.int32(_D)).reshape(1)
    fusion_feature, idx8 = _sort_topk_fusion(s2d, tko, u, Wv, bv.reshape(1, _D))
    idx_flat = idx8.reshape(-1)
    top_k_features = _make_gather_sc()(feats, idx_flat)
    return (top_k_features, fusion_feature)


# speedup vs baseline: 2.6578x; 1.2816x over previous
import functools

import jax
import jax.numpy as jnp
import numpy as np
from jax import lax
from jax.experimental import pallas as pl
from jax.experimental.pallas import tpu as pltpu
from jax.experimental.pallas import tpu_sc as plsc

_N = 16384
_D = 1024
_DQ = 128
_KC = 256
_RSCALE = np.float32(1.0) / np.sqrt(np.float32(128.0))

_bf16 = jnp.bfloat16
_f32 = jnp.float32


_MBLK = 2048


def _score_body(kf_ref, wq_ref, bq_ref, f_ref, s_ref, u_ref, qkb_sc, m_sc):
    i = pl.program_id(0)

    @pl.when(i == 0)
    def _qk():
        qk = jnp.dot(kf_ref[...], wq_ref[...]) + bq_ref[...]
        qkb_sc[...] = qk.astype(_bf16)

    acc = jnp.zeros((_MBLK, _DQ), _f32)
    for j in range(_D // _KC):
        acc = acc + jnp.dot(f_ref[:, j * _KC:(j + 1) * _KC],
                            wq_ref[j * _KC:(j + 1) * _KC, :])
    qb = (acc + bq_ref[...]).astype(_bf16)
    s_blk = jnp.dot(qb, qkb_sc[...].T, preferred_element_type=_f32)
    s_ref[...] = s_blk

    x = s_blk * _RSCALE
    bmax = jnp.max(x)

    @pl.when(i == 0)
    def _init():
        m_sc[0] = -jnp.inf
        u_ref[...] = jnp.zeros_like(u_ref)

    m_old = m_sc[0]
    m_new = jnp.maximum(m_old, bmax)
    e = jnp.exp(x - m_new)
    part = lax.dot_general(e, f_ref[...],
                           dimension_numbers=(((0,), (0,)), ((), ())),
                           preferred_element_type=_f32)
    u_ref[...] = u_ref[...] * jnp.exp(m_old - m_new) + part
    m_sc[0] = m_new


def _scores(key_feat, Wq, bq2, feats):
    return pl.pallas_call(
        _score_body,
        grid=(_N // _MBLK,),
        in_specs=[
            pl.BlockSpec((1, _D), lambda i: (0, 0)),
            pl.BlockSpec((_D, _DQ), lambda i: (0, 0)),
            pl.BlockSpec((1, _DQ), lambda i: (0, 0)),
            pl.BlockSpec((_MBLK, _D), lambda i: (i, 0)),
        ],
        out_specs=[
            pl.BlockSpec((_MBLK, 1), lambda i: (i, 0)),
            pl.BlockSpec((1, _D), lambda i: (0, 0)),
        ],
        out_shape=[
            jax.ShapeDtypeStruct((_N, 1), _f32),
            jax.ShapeDtypeStruct((1, _D), _f32),
        ],
        scratch_shapes=[pltpu.VMEM((1, _DQ), _bf16), pltpu.SMEM((1,), _f32)],
    )(key_feat, Wq, bq2, feats)


def _t_i32(x):
    return lax.bitcast_convert_type(
        lax.bitcast_convert_type(x, _f32).T, jnp.int32)


def _bitonic_topk(key, idx, trans):
    for p in range(1, 15):
        for j in reversed(range(p)):
            need_trans = j >= 7
            if need_trans != trans:
                key = key.T
                idx = _t_i32(idx)
                trans = need_trans
            sh = (1 << j) if j < 7 else (1 << (j - 7))
            io0 = lax.broadcasted_iota(jnp.int32, (128, 128), 0)
            up = (io0 & sh) == 0
            if p < 7:
                ax = 1 if trans else 0
                bit = 1 << p
            else:
                ax = 0 if trans else 1
                bit = 1 << (p - 7)
            iod = lax.broadcasted_iota(jnp.int32, (128, 128), ax)
            dirm = (iod & bit) == 0
            pk = jnp.where(up, jnp.roll(key, -sh, axis=0), jnp.roll(key, sh, axis=0))
            pi = jnp.where(up, jnp.roll(idx, -sh, axis=0), jnp.roll(idx, sh, axis=0))
            better = (key > pk) | ((key == pk) & (idx < pi))
            keep = (up == better) == dirm
            key = jnp.where(keep, key, pk)
            idx = jnp.where(keep, idx, pi)
    if trans:
        key = key.T
        idx = _t_i32(idx)
    return key, idx


def _sort_body(s_ref, tko_ref, u_ref, wv_ref, bv_ref, fus_ref, idx_ref):
    v = s_ref[...]
    x = v * _RSCALE
    m = jnp.max(x)
    e = jnp.exp(x - m)
    z = jnp.sum(e)
    a = e / z
    a_row = u_ref[...] / z
    fus_ref[...] = jnp.dot(a_row, wv_ref[...], precision="highest",
                           preferred_element_type=_f32) + bv_ref[...]
    hi_io = lax.broadcasted_iota(jnp.int32, (128, 128), 0)
    lo_io = lax.broadcasted_iota(jnp.int32, (128, 128), 1)
    n = 128 * hi_io + lo_io
    _, idx_sorted = _bitonic_topk(a, n, True)
    top = idx_sorted[:, 0:8]
    idx_ref[...] = _t_i32(top) + tko_ref[0]


def _sort_topk_fusion(s2d, tko, u, Wv, bv2):
    return pl.pallas_call(
        _sort_body,
        in_specs=[
            pl.BlockSpec((128, 128), lambda: (0, 0)),
            pl.BlockSpec(memory_space=pltpu.SMEM),
            pl.BlockSpec((1, _D), lambda: (0, 0)),
            pl.BlockSpec((_D, _D), lambda: (0, 0)),
            pl.BlockSpec((1, _D), lambda: (0, 0)),
        ],
        out_specs=[
            pl.BlockSpec((1, _D), lambda: (0, 0)),
            pl.BlockSpec((8, 128), lambda: (0, 0)),
        ],
        out_shape=[
            jax.ShapeDtypeStruct((1, _D), _f32),
            jax.ShapeDtypeStruct((8, 128), jnp.int32),
        ],
    )(s2d, tko, u, Wv, bv2)


_NW = 32
_BPW = 1024 // _NW


@functools.lru_cache(maxsize=1)
def _make_gather_sc():
    @functools.partial(
        pl.kernel,
        out_type=jax.ShapeDtypeStruct((1024, _D), _f32),
        mesh=plsc.VectorSubcoreMesh(core_axis_name="c", subcore_axis_name="s"),
        scratch_types=[
            pltpu.VMEM((_BPW,), jnp.int32),
            pltpu.VMEM((_BPW, _D), _f32),
            pltpu.SemaphoreType.DMA,
        ],
    )
    def _gather_sc(feats_hbm, idx_hbm, out_hbm, idx_v, rows_v, sem):
        wid = lax.axis_index("s") * 2 + lax.axis_index("c")
        base = wid * _BPW
        pltpu.sync_copy(idx_hbm.at[pl.ds(base, _BPW)], idx_v)
        pltpu.async_copy(feats_hbm.at[idx_v], rows_v, sem).wait()
        pltpu.sync_copy(rows_v, out_hbm.at[pl.ds(base, _BPW)])

    return _gather_sc


def kernel(feats, key_feat, Wq, bq, Wv, bv, top_k):
    bq2 = bq.reshape(1, _DQ)
    s, u = _scores(key_feat, Wq, bq2, feats)
    s2d = s.reshape(128, 128)
    tko = (jnp.asarray(top_k, jnp.int32) - jnp.int32(_D)).reshape(1)
    fusion_feature, idx8 = _sort_topk_fusion(s2d, tko, u, Wv, bv.reshape(1, _D))
    idx_flat = idx8.reshape(-1)
    top_k_features = _make_gather_sc()(feats, idx_flat)
    return (top_k_features, fusion_feature)


# speedup vs baseline: 2.6856x; 1.0105x over previous
import functools

import jax
import jax.numpy as jnp
import numpy as np
from jax import lax
from jax.experimental import pallas as pl
from jax.experimental.pallas import tpu as pltpu
from jax.experimental.pallas import tpu_sc as plsc

_N = 16384
_D = 1024
_DQ = 128
_KC = 256
_RSCALE = np.float32(1.0) / np.sqrt(np.float32(128.0))

_bf16 = jnp.bfloat16
_f32 = jnp.float32


_MBLK = 4096


def _score_body(kf_ref, wq_ref, bq_ref, f_ref, s_ref, u_ref, qkb_sc, m_sc):
    i = pl.program_id(0)

    @pl.when(i == 0)
    def _qk():
        qk = jnp.dot(kf_ref[...], wq_ref[...]) + bq_ref[...]
        qkb_sc[...] = qk.astype(_bf16)

    acc = jnp.zeros((_MBLK, _DQ), _f32)
    for j in range(_D // _KC):
        acc = acc + jnp.dot(f_ref[:, j * _KC:(j + 1) * _KC],
                            wq_ref[j * _KC:(j + 1) * _KC, :])
    qb = (acc + bq_ref[...]).astype(_bf16)
    s_blk = jnp.dot(qb, qkb_sc[...].T, preferred_element_type=_f32)
    s_ref[...] = s_blk

    x = s_blk * _RSCALE
    bmax = jnp.max(x)

    @pl.when(i == 0)
    def _init():
        m_sc[0] = -jnp.inf
        u_ref[...] = jnp.zeros_like(u_ref)

    m_old = m_sc[0]
    m_new = jnp.maximum(m_old, bmax)
    e = jnp.exp(x - m_new)
    part = lax.dot_general(e, f_ref[...],
                           dimension_numbers=(((0,), (0,)), ((), ())),
                           preferred_element_type=_f32)
    u_ref[...] = u_ref[...] * jnp.exp(m_old - m_new) + part
    m_sc[0] = m_new


def _scores(key_feat, Wq, bq2, feats):
    return pl.pallas_call(
        _score_body,
        grid=(_N // _MBLK,),
        in_specs=[
            pl.BlockSpec((1, _D), lambda i: (0, 0)),
            pl.BlockSpec((_D, _DQ), lambda i: (0, 0)),
            pl.BlockSpec((1, _DQ), lambda i: (0, 0)),
            pl.BlockSpec((_MBLK, _D), lambda i: (i, 0)),
        ],
        out_specs=[
            pl.BlockSpec((_MBLK, 1), lambda i: (i, 0)),
            pl.BlockSpec((1, _D), lambda i: (0, 0)),
        ],
        out_shape=[
            jax.ShapeDtypeStruct((_N, 1), _f32),
            jax.ShapeDtypeStruct((1, _D), _f32),
        ],
        scratch_shapes=[pltpu.VMEM((1, _DQ), _bf16), pltpu.SMEM((1,), _f32)],
    )(key_feat, Wq, bq2, feats)


def _t_i32(x):
    return lax.bitcast_convert_type(
        lax.bitcast_convert_type(x, _f32).T, jnp.int32)


def _bitonic_topk(key, idx, trans):
    for p in range(1, 15):
        for j in reversed(range(p)):
            need_trans = j >= 7
            if need_trans != trans:
                key = key.T
                idx = _t_i32(idx)
                trans = need_trans
            sh = (1 << j) if j < 7 else (1 << (j - 7))
            io0 = lax.broadcasted_iota(jnp.int32, (128, 128), 0)
            up = (io0 & sh) == 0
            if p < 7:
                ax = 1 if trans else 0
                bit = 1 << p
            else:
                ax = 0 if trans else 1
                bit = 1 << (p - 7)
            iod = lax.broadcasted_iota(jnp.int32, (128, 128), ax)
            dirm = (iod & bit) == 0
            pk = jnp.where(up, jnp.roll(key, -sh, axis=0), jnp.roll(key, sh, axis=0))
            pi = jnp.where(up, jnp.roll(idx, -sh, axis=0), jnp.roll(idx, sh, axis=0))
            better = (key > pk) | ((key == pk) & (idx < pi))
            keep = (up == better) == dirm
            key = jnp.where(keep, key, pk)
            idx = jnp.where(keep, idx, pi)
    if trans:
        key = key.T
        idx = _t_i32(idx)
    return key, idx


def _sort_body(s_ref, tko_ref, u_ref, wv_ref, bv_ref, fus_ref, idx_ref):
    v = s_ref[...]
    x = v * _RSCALE
    m = jnp.max(x)
    e = jnp.exp(x - m)
    z = jnp.sum(e)
    a = e / z
    a_row = u_ref[...] / z
    fus_ref[...] = jnp.dot(a_row, wv_ref[...], precision="highest",
                           preferred_element_type=_f32) + bv_ref[...]
    hi_io = lax.broadcasted_iota(jnp.int32, (128, 128), 0)
    lo_io = lax.broadcasted_iota(jnp.int32, (128, 128), 1)
    n = 128 * hi_io + lo_io
    _, idx_sorted = _bitonic_topk(a, n, True)
    top = idx_sorted[:, 0:8]
    idx_ref[...] = _t_i32(top) + tko_ref[0]


def _sort_topk_fusion(s2d, tko, u, Wv, bv2):
    return pl.pallas_call(
        _sort_body,
        in_specs=[
            pl.BlockSpec((128, 128), lambda: (0, 0)),
            pl.BlockSpec(memory_space=pltpu.SMEM),
            pl.BlockSpec((1, _D), lambda: (0, 0)),
            pl.BlockSpec((_D, _D), lambda: (0, 0)),
            pl.BlockSpec((1, _D), lambda: (0, 0)),
        ],
        out_specs=[
            pl.BlockSpec((1, _D), lambda: (0, 0)),
            pl.BlockSpec((8, 128), lambda: (0, 0)),
        ],
        out_shape=[
            jax.ShapeDtypeStruct((1, _D), _f32),
            jax.ShapeDtypeStruct((8, 128), jnp.int32),
        ],
    )(s2d, tko, u, Wv, bv2)


_NW = 32
_BPW = 1024 // _NW


@functools.lru_cache(maxsize=1)
def _make_gather_sc():
    @functools.partial(
        pl.kernel,
        out_type=jax.ShapeDtypeStruct((1024, _D), _f32),
        mesh=plsc.VectorSubcoreMesh(core_axis_name="c", subcore_axis_name="s"),
        scratch_types=[
            pltpu.VMEM((_BPW,), jnp.int32),
            pltpu.VMEM((_BPW, _D), _f32),
            pltpu.SemaphoreType.DMA,
        ],
    )
    def _gather_sc(feats_hbm, idx_hbm, out_hbm, idx_v, rows_v, sem):
        wid = lax.axis_index("s") * 2 + lax.axis_index("c")
        base = wid * _BPW
        pltpu.sync_copy(idx_hbm.at[pl.ds(base, _BPW)], idx_v)
        pltpu.async_copy(feats_hbm.at[idx_v], rows_v, sem).wait()
        pltpu.sync_copy(rows_v, out_hbm.at[pl.ds(base, _BPW)])

    return _gather_sc


def kernel(feats, key_feat, Wq, bq, Wv, bv, top_k):
    bq2 = bq.reshape(1, _DQ)
    s, u = _scores(key_feat, Wq, bq2, feats)
    s2d = s.reshape(128, 128)
    tko = (jnp.asarray(top_k, jnp.int32) - jnp.int32(_D)).reshape(1)
    fusion_feature, idx8 = _sort_topk_fusion(s2d, tko, u, Wv, bv.reshape(1, _D))
    idx_flat = idx8.reshape(-1)
    top_k_features = _make_gather_sc()(feats, idx_flat)
    return (top_k_features, fusion_feature)
